# Initial kernel scaffold; baseline (speedup 1.0000x reference)
import jax
import jax.numpy as jnp
from jax.experimental import pallas as pl


def kernel(x, graph_edge_index, edge_pairs, W1, b1, W2, b2, Wm1, bm1, Wm2, bm2):
    raise NotImplementedError("write your pallas kernel here")


# speedup vs baseline: 14.0820x; 14.0820x over previous
import functools

import jax
import jax.numpy as jnp
from jax import lax
from jax.experimental import pallas as pl
from jax.experimental.pallas import tpu as pltpu
from jax.experimental.pallas import tpu_sc as plsc

N = 10000
D = 128
H = 16
E = 320000
EP = 200000
M = 32

NT = 32
CH = 128
KE = 80
EPAD = NT * KE * CH
NP = 10240
RPT = NP // 16
KUV = 104
EPP = NT * KUV * CH // 2
UVTOT = NT * KUV * CH

_mesh = plsc.VectorSubcoreMesh(core_axis_name="c", subcore_axis_name="s")


@functools.partial(
    pl.kernel,
    mesh=_mesh,
    out_type=jax.ShapeDtypeStruct((2, NP), jnp.float32),
    compiler_params=pltpu.CompilerParams(use_tc_tiling_on_sc=False),
    scratch_types=[
        pltpu.VMEM((KE, CH), jnp.int32),
        pltpu.VMEM((CH,), jnp.float32),
        pltpu.VMEM_SHARED((NP,), jnp.float32),
    ],
)
def _deg_kernel(dst_hbm, zeros_hbm, out_hbm, idx_v, ones_v, deg_sh):
    c = lax.axis_index("c")
    s = lax.axis_index("s")
    wid = c * 16 + s
    pltpu.sync_copy(zeros_hbm.at[pl.ds(s * RPT, RPT)],
                    deg_sh.at[pl.ds(s * RPT, RPT)])
    for i in range(CH // 16):
        ones_v[pl.ds(i * 16, 16)] = jnp.ones((16,), jnp.float32)
    pltpu.sync_copy(dst_hbm.at[pl.ds(wid * KE, KE)], idx_v)
    plsc.subcore_barrier()

    def body(j, carry):
        pltpu.sync_copy(ones_v, deg_sh.at[idx_v.at[j]], add=True)
        return carry

    lax.fori_loop(0, KE, body, 0)
    plsc.subcore_barrier()
    pltpu.sync_copy(deg_sh.at[pl.ds(s * RPT, RPT)],
                    out_hbm.at[c, pl.ds(s * RPT, RPT)])


@functools.partial(
    pl.kernel,
    mesh=_mesh,
    out_type=jax.ShapeDtypeStruct((2, NP, H), jnp.float32),
    compiler_params=pltpu.CompilerParams(use_tc_tiling_on_sc=False),
    scratch_types=[
        pltpu.VMEM((KE, CH), jnp.int32),
        pltpu.VMEM((KE, CH), jnp.int32),
        pltpu.VMEM((CH, H), jnp.float32),
        pltpu.VMEM_SHARED((NP, H), jnp.float32),
        pltpu.SemaphoreType.DMA,
    ],
)
def _agg_kernel(src_hbm, dst_hbm, y_hbm, zeros_hbm, out_hbm,
                src_v, dst_v, buf, acc_sh, sem):
    c = lax.axis_index("c")
    s = lax.axis_index("s")
    wid = c * 16 + s
    pltpu.sync_copy(zeros_hbm.at[pl.ds(s * RPT, RPT)],
                    acc_sh.at[pl.ds(s * RPT, RPT)])
    pltpu.sync_copy(src_hbm.at[pl.ds(wid * KE, KE)], src_v)
    pltpu.sync_copy(dst_hbm.at[pl.ds(wid * KE, KE)], dst_v)
    plsc.subcore_barrier()

    def body(j, carry):
        pltpu.async_copy(y_hbm.at[src_v.at[j]], buf, sem).wait()
        pltpu.sync_copy(buf, acc_sh.at[dst_v.at[j]], add=True)
        return carry

    lax.fori_loop(0, KE, body, 0)
    plsc.subcore_barrier()
    pltpu.sync_copy(acc_sh.at[pl.ds(s * RPT, RPT)],
                    out_hbm.at[c, pl.ds(s * RPT, RPT)])


@functools.partial(
    pl.kernel,
    mesh=_mesh,
    out_type=jax.ShapeDtypeStruct((UVTOT, H), jnp.float32),
    compiler_params=pltpu.CompilerParams(use_tc_tiling_on_sc=False),
    scratch_types=[
        pltpu.VMEM((KUV, CH), jnp.int32),
        pltpu.VMEM((CH, H), jnp.float32),
        pltpu.SemaphoreType.DMA,
    ],
)
def _pairs_gather_kernel(idx_hbm, h_hbm, out_hbm, idx_v, buf, sem):
    c = lax.axis_index("c")
    s = lax.axis_index("s")
    wid = c * 16 + s
    base = wid * KUV
    pltpu.sync_copy(idx_hbm.at[pl.ds(base, KUV)], idx_v)

    def body(j, carry):
        pltpu.async_copy(h_hbm.at[idx_v.at[j]], buf, sem).wait()
        pltpu.sync_copy(buf, out_hbm.at[pl.ds((base + j) * CH, CH)])
        return carry

    lax.fori_loop(0, KUV, body, 0)


def _mm1_body(x_ref, w_ref, dis_ref, y_ref):
    y_ref[...] = jnp.dot(x_ref[...], w_ref[...],
                         preferred_element_type=jnp.float32) * dis_ref[...]


def _mid_body(accp_ref, y1_ref, dis_ref, b1_ref, w2_ref, y2_ref):
    acc = accp_ref[0] + accp_ref[1] + y1_ref[...]
    h1 = jnp.maximum(acc * dis_ref[...] + b1_ref[...], 0.0)
    rows = lax.broadcasted_iota(jnp.int32, (NP, 1), 0)
    h1 = jnp.where(rows < N, h1, 0.0)
    y2_ref[...] = jnp.dot(h1, w2_ref[...],
                          preferred_element_type=jnp.float32) * dis_ref[...]


def _fin_body(accp_ref, y2_ref, dis_ref, b2_ref, h2_ref):
    acc = accp_ref[0] + accp_ref[1] + y2_ref[...]
    h2_ref[...] = jnp.maximum(acc * dis_ref[...] + b2_ref[...], 0.0)


def _head_body(u_ref, v_ref, wa_ref, wb_ref, wc_ref, wd_ref, bm1_ref,
               wm2_ref, bm2_ref, o_ref):
    u = u_ref[...]
    v = v_ref[...]
    hid = (jnp.dot(u, wa_ref[...], preferred_element_type=jnp.float32)
           + jnp.dot(v, wb_ref[...], preferred_element_type=jnp.float32)
           + jnp.dot(jnp.abs(u - v), wc_ref[...],
                     preferred_element_type=jnp.float32)
           + jnp.dot(u * v, wd_ref[...], preferred_element_type=jnp.float32)
           + bm1_ref[...])
    hid = jnp.maximum(hid, 0.0)
    o_ref[...] = jnp.dot(hid, wm2_ref[...],
                         preferred_element_type=jnp.float32) + bm2_ref[...]


BH = 6656


def kernel(x, graph_edge_index, edge_pairs, W1, b1, W2, b2, Wm1, bm1, Wm2, bm2):
    src = graph_edge_index[0]
    dst = graph_edge_index[1]
    pad_e = jnp.full((EPAD - E,), N, jnp.int32)
    srcp = jnp.concatenate([src, pad_e]).reshape(EPAD // CH, CH)
    dstp = jnp.concatenate([dst, pad_e]).reshape(EPAD // CH, CH)
    zeros1d = jnp.zeros((NP,), jnp.float32)
    zeros2d = jnp.zeros((NP, H), jnp.float32)
    x_pad = jnp.concatenate([x, jnp.zeros((NP - N, D), jnp.float32)], axis=0)

    degp = _deg_kernel(dstp, zeros1d)
    deg = degp[0] + degp[1] + 1.0
    dis = jnp.where(deg > 0, deg ** -0.5, 0.0)
    dis_col = dis[:, None]

    y1 = pl.pallas_call(
        _mm1_body,
        out_shape=jax.ShapeDtypeStruct((NP, H), jnp.float32),
    )(x_pad, W1, dis_col)

    accp1 = _agg_kernel(srcp, dstp, y1, zeros2d)

    y2 = pl.pallas_call(
        _mid_body,
        out_shape=jax.ShapeDtypeStruct((NP, H), jnp.float32),
    )(accp1, y1, dis_col, b1.reshape(1, H), W2)

    accp2 = _agg_kernel(srcp, dstp, y2, zeros2d)

    h2 = pl.pallas_call(
        _fin_body,
        out_shape=jax.ShapeDtypeStruct((NP, H), jnp.float32),
    )(accp2, y2, dis_col, b2.reshape(1, H))

    pad_p = jnp.zeros((EPP - EP,), jnp.int32)
    uv_idx = jnp.concatenate(
        [edge_pairs[0], pad_p, edge_pairs[1], pad_p]).reshape(UVTOT // CH, CH)
    uv = _pairs_gather_kernel(uv_idx, h2)

    nb = EPP // BH
    out = pl.pallas_call(
        _head_body,
        grid=(nb,),
        in_specs=[
            pl.BlockSpec((BH, H), lambda i: (i, 0)),
            pl.BlockSpec((BH, H), lambda i: (i + nb, 0)),
            pl.BlockSpec((H, M), lambda i: (0, 0)),
            pl.BlockSpec((H, M), lambda i: (0, 0)),
            pl.BlockSpec((H, M), lambda i: (0, 0)),
            pl.BlockSpec((H, M), lambda i: (0, 0)),
            pl.BlockSpec((1, M), lambda i: (0, 0)),
            pl.BlockSpec((M, 1), lambda i: (0, 0)),
            pl.BlockSpec((1, 1), lambda i: (0, 0)),
        ],
        out_specs=pl.BlockSpec((BH, 1), lambda i: (i, 0)),
        out_shape=jax.ShapeDtypeStruct((EPP, 1), jnp.float32),
    )(uv, uv, Wm1[0:H], Wm1[H:2 * H], Wm1[2 * H:3 * H], Wm1[3 * H:4 * H],
      bm1.reshape(1, M), Wm2, bm2.reshape(1, 1))
    return out[:EP, 0]


# speedup vs baseline: 23.9487x; 1.7007x over previous
import functools

import jax
import jax.numpy as jnp
from jax import lax
from jax.experimental import pallas as pl
from jax.experimental.pallas import tpu as pltpu
from jax.experimental.pallas import tpu_sc as plsc

N = 10000
D = 128
H = 16
E = 320000
EP = 200000
M = 32

NT = 32
CH = 128
KE = 80
EPAD = NT * KE * CH
NP = 10240
RPT = NP // 16
KUV = 104
EPP = NT * KUV * CH // 2
UVTOT = NT * KUV * CH

_mesh = plsc.VectorSubcoreMesh(core_axis_name="c", subcore_axis_name="s")


@functools.partial(
    pl.kernel,
    mesh=_mesh,
    out_type=jax.ShapeDtypeStruct((2, NP), jnp.float32),
    compiler_params=pltpu.CompilerParams(use_tc_tiling_on_sc=False),
    scratch_types=[
        pltpu.VMEM((KE, CH), jnp.int32),
        pltpu.VMEM((CH,), jnp.float32),
        pltpu.VMEM_SHARED((NP,), jnp.float32),
    ],
)
def _deg_kernel(dst_hbm, zeros_hbm, out_hbm, idx_v, ones_v, deg_sh):
    c = lax.axis_index("c")
    s = lax.axis_index("s")
    wid = c * 16 + s
    pltpu.sync_copy(zeros_hbm.at[pl.ds(s * RPT, RPT)],
                    deg_sh.at[pl.ds(s * RPT, RPT)])
    for i in range(CH // 16):
        ones_v[pl.ds(i * 16, 16)] = jnp.ones((16,), jnp.float32)
    pltpu.sync_copy(dst_hbm.at[pl.ds(wid * KE, KE)], idx_v)
    plsc.subcore_barrier()

    def body(j, carry):
        pltpu.sync_copy(ones_v, deg_sh.at[idx_v.at[j]], add=True)
        return carry

    lax.fori_loop(0, KE, body, 0)
    plsc.subcore_barrier()
    pltpu.sync_copy(deg_sh.at[pl.ds(s * RPT, RPT)],
                    out_hbm.at[c, pl.ds(s * RPT, RPT)])


GQ = 4
NG_E = KE // GQ


@functools.partial(
    pl.kernel,
    mesh=_mesh,
    out_type=jax.ShapeDtypeStruct((2, NP, H), jnp.float32),
    compiler_params=pltpu.CompilerParams(use_tc_tiling_on_sc=False),
    scratch_types=[
        pltpu.VMEM((KE, CH), jnp.int32),
        pltpu.VMEM((KE, CH), jnp.int32),
        pltpu.VMEM((GQ * CH, H), jnp.float32),
        pltpu.VMEM((GQ * CH, H), jnp.float32),
        pltpu.VMEM_SHARED((NP, H), jnp.float32),
        pltpu.SemaphoreType.DMA,
        pltpu.SemaphoreType.DMA,
    ],
)
def _agg_kernel(src_hbm, dst_hbm, y_hbm, zeros_hbm, out_hbm,
                src_v, dst_v, buf_a, buf_b, acc_sh, sem_a, sem_b):
    c = lax.axis_index("c")
    s = lax.axis_index("s")
    wid = c * 16 + s
    pltpu.sync_copy(zeros_hbm.at[pl.ds(s * RPT, RPT)],
                    acc_sh.at[pl.ds(s * RPT, RPT)])
    pltpu.sync_copy(src_hbm.at[pl.ds(wid * KE, KE)], src_v)
    pltpu.sync_copy(dst_hbm.at[pl.ds(wid * KE, KE)], dst_v)
    plsc.subcore_barrier()

    def fire(g, buf, sem):
        for q in range(GQ):
            pltpu.async_copy(y_hbm.at[src_v.at[g * GQ + q]],
                             buf.at[pl.ds(q * CH, CH)], sem)

    def drain(g, buf, sem):
        for q in range(GQ):
            pltpu.make_async_copy(y_hbm.at[src_v.at[g * GQ + q]],
                                  buf.at[pl.ds(q * CH, CH)], sem).wait()

    def scat(g, buf):
        for q in range(GQ):
            pltpu.sync_copy(buf.at[pl.ds(q * CH, CH)],
                            acc_sh.at[dst_v.at[g * GQ + q]], add=True)

    fire(0, buf_a, sem_a)

    def body(ii, carry):
        g0 = 2 * ii
        g1 = g0 + 1
        fire(g1, buf_b, sem_b)
        drain(g0, buf_a, sem_a)
        scat(g0, buf_a)

        @pl.when(g0 + 2 < NG_E)
        def _():
            fire(g0 + 2, buf_a, sem_a)

        drain(g1, buf_b, sem_b)
        scat(g1, buf_b)
        return carry

    lax.fori_loop(0, NG_E // 2, body, 0)
    plsc.subcore_barrier()
    pltpu.sync_copy(acc_sh.at[pl.ds(s * RPT, RPT)],
                    out_hbm.at[c, pl.ds(s * RPT, RPT)])


NG_P = KUV // GQ


@functools.partial(
    pl.kernel,
    mesh=_mesh,
    out_type=jax.ShapeDtypeStruct((UVTOT, H), jnp.float32),
    compiler_params=pltpu.CompilerParams(use_tc_tiling_on_sc=False),
    scratch_types=[
        pltpu.VMEM((KUV, CH), jnp.int32),
        pltpu.VMEM((GQ * CH, H), jnp.float32),
        pltpu.VMEM((GQ * CH, H), jnp.float32),
        pltpu.SemaphoreType.DMA,
        pltpu.SemaphoreType.DMA,
        pltpu.SemaphoreType.DMA,
        pltpu.SemaphoreType.DMA,
    ],
)
def _pairs_gather_kernel(idx_hbm, h_hbm, out_hbm, idx_v, buf_a, buf_b,
                         sga, sgb, swa, swb):
    c = lax.axis_index("c")
    s = lax.axis_index("s")
    wid = c * 16 + s
    base = wid * KUV
    pltpu.sync_copy(idx_hbm.at[pl.ds(wid * KUV, KUV)], idx_v)

    def fire(g, buf, sem):
        for q in range(GQ):
            pltpu.async_copy(h_hbm.at[idx_v.at[g * GQ + q]],
                             buf.at[pl.ds(q * CH, CH)], sem)

    def drain(g, buf, sem):
        for q in range(GQ):
            pltpu.make_async_copy(h_hbm.at[idx_v.at[g * GQ + q]],
                                  buf.at[pl.ds(q * CH, CH)], sem).wait()

    def wstart(g, buf, sem):
        pltpu.async_copy(buf, out_hbm.at[pl.ds((base + g * GQ) * CH, GQ * CH)],
                         sem)

    def wwait(g, buf, sem):
        pltpu.make_async_copy(buf,
                              out_hbm.at[pl.ds((base + g * GQ) * CH, GQ * CH)],
                              sem).wait()

    fire(0, buf_a, sga)
    fire(1, buf_b, sgb)

    def body(ii, carry):
        g0 = 2 * ii
        g1 = g0 + 1
        drain(g0, buf_a, sga)
        wstart(g0, buf_a, swa)
        drain(g1, buf_b, sgb)
        wstart(g1, buf_b, swb)

        @pl.when(g0 + 2 < NG_P)
        def _():
            wwait(g0, buf_a, swa)
            fire(g0 + 2, buf_a, sga)

        @pl.when(g1 + 2 < NG_P)
        def _():
            wwait(g1, buf_b, swb)
            fire(g1 + 2, buf_b, sgb)

        return carry

    lax.fori_loop(0, NG_P // 2, body, 0)
    wwait(NG_P - 2, buf_a, swa)
    wwait(NG_P - 1, buf_b, swb)


def _mm1_body(x_ref, w_ref, dis_ref, y_ref):
    y_ref[...] = jnp.dot(x_ref[...], w_ref[...],
                         preferred_element_type=jnp.float32) * dis_ref[...]


def _mid_body(accp_ref, y1_ref, dis_ref, b1_ref, w2_ref, y2_ref):
    acc = accp_ref[0] + accp_ref[1] + y1_ref[...]
    h1 = jnp.maximum(acc * dis_ref[...] + b1_ref[...], 0.0)
    rows = lax.broadcasted_iota(jnp.int32, (NP, 1), 0)
    h1 = jnp.where(rows < N, h1, 0.0)
    y2_ref[...] = jnp.dot(h1, w2_ref[...],
                          preferred_element_type=jnp.float32) * dis_ref[...]


def _fin_body(accp_ref, y2_ref, dis_ref, b2_ref, h2_ref):
    acc = accp_ref[0] + accp_ref[1] + y2_ref[...]
    h2_ref[...] = jnp.maximum(acc * dis_ref[...] + b2_ref[...], 0.0)


def _head_body(u_ref, v_ref, w1_ref, b1_ref, w2_ref, b2_ref, o_ref):
    u = u_ref[...]
    v = v_ref[...]
    feat = jnp.concatenate([u, v, jnp.abs(u - v), u * v], axis=1)
    hid = jnp.maximum(
        jnp.dot(feat, w1_ref[...], preferred_element_type=jnp.float32)
        + b1_ref[...], 0.0)
    o_ref[...] = jnp.dot(hid, w2_ref[...],
                         preferred_element_type=jnp.float32) + b2_ref[...]


PR = EPP // 8
NBH = 32
RB = PR // NBH


def kernel(x, graph_edge_index, edge_pairs, W1, b1, W2, b2, Wm1, bm1, Wm2, bm2):
    src = graph_edge_index[0]
    dst = graph_edge_index[1]
    pad_e = jnp.full((EPAD - E,), N, jnp.int32)
    srcp = jnp.concatenate([src, pad_e]).reshape(EPAD // CH, CH)
    dstp = jnp.concatenate([dst, pad_e]).reshape(EPAD // CH, CH)
    zeros1d = jnp.zeros((NP,), jnp.float32)
    zeros2d = jnp.zeros((NP, H), jnp.float32)
    x_pad = jnp.concatenate([x, jnp.zeros((NP - N, D), jnp.float32)], axis=0)

    degp = _deg_kernel(dstp, zeros1d)
    deg = degp[0] + degp[1] + 1.0
    dis = jnp.where(deg > 0, deg ** -0.5, 0.0)
    dis_col = dis[:, None]

    y1 = pl.pallas_call(
        _mm1_body,
        out_shape=jax.ShapeDtypeStruct((NP, H), jnp.float32),
    )(x_pad, W1, dis_col)

    accp1 = _agg_kernel(srcp, dstp, y1, zeros2d)

    y2 = pl.pallas_call(
        _mid_body,
        out_shape=jax.ShapeDtypeStruct((NP, H), jnp.float32),
    )(accp1, y1, dis_col, b1.reshape(1, H), W2)

    accp2 = _agg_kernel(srcp, dstp, y2, zeros2d)

    h2 = pl.pallas_call(
        _fin_body,
        out_shape=jax.ShapeDtypeStruct((NP, H), jnp.float32),
    )(accp2, y2, dis_col, b2.reshape(1, H))

    pad_p = jnp.zeros((EPP - EP,), jnp.int32)
    uv_idx = jnp.concatenate(
        [edge_pairs[0], pad_p, edge_pairs[1], pad_p]).reshape(UVTOT // CH, CH)
    uv = _pairs_gather_kernel(uv_idx, h2)
    uvp = uv.reshape(UVTOT // 8, 128)

    eye8 = jnp.eye(8, dtype=jnp.float32)
    w1_big = jnp.concatenate([
        jnp.kron(eye8, Wm1[0:H]),
        jnp.kron(eye8, Wm1[H:2 * H]),
        jnp.kron(eye8, Wm1[2 * H:3 * H]),
        jnp.kron(eye8, Wm1[3 * H:4 * H]),
    ], axis=0)
    w2_big = jnp.kron(eye8, Wm2)
    b1_big = jnp.tile(bm1, 8).reshape(1, 8 * M)
    b2_big = jnp.tile(bm2, 8).reshape(1, 8)

    out = pl.pallas_call(
        _head_body,
        grid=(NBH,),
        in_specs=[
            pl.BlockSpec((RB, 128), lambda i: (i, 0)),
            pl.BlockSpec((RB, 128), lambda i: (i + NBH, 0)),
            pl.BlockSpec((4 * 128, 8 * M), lambda i: (0, 0)),
            pl.BlockSpec((1, 8 * M), lambda i: (0, 0)),
            pl.BlockSpec((8 * M, 8), lambda i: (0, 0)),
            pl.BlockSpec((1, 8), lambda i: (0, 0)),
        ],
        out_specs=pl.BlockSpec((RB, 8), lambda i: (i, 0)),
        out_shape=jax.ShapeDtypeStruct((PR, 8), jnp.float32),
    )(uvp, uvp, w1_big, b1_big, w2_big, b2_big)
    return out.reshape(-1)[:EP]
